# Initial kernel scaffold; baseline (speedup 1.0000x reference)
import jax
import jax.numpy as jnp
from jax.experimental import pallas as pl


def kernel(x, edge_index, W_in, b_in, eW1, eb1, eW2, eb2, eW3, eb3, eW4, eb4, nW1, nb1, nW2, nb2, nW3, nb3, nW4, nb4):
    raise NotImplementedError("write your pallas kernel here")


# speedup vs baseline: 2.2992x; 2.2992x over previous
import functools

import jax
import jax.numpy as jnp
from jax import lax
from jax.experimental import pallas as pl
from jax.experimental.pallas import tpu as pltpu
from jax.experimental.pallas import tpu_sc as plsc

N = 10000
E = 320000
H = 128

NC = 2
NS = 16
EPT = E // NS
BG = 80
NB = EPT // BG
NPAD = 10240
ROWS_PT = NPAD // NS

@functools.lru_cache(maxsize=None)
def _get_mesh():
    return plsc.VectorSubcoreMesh(
        core_axis_name="c", subcore_axis_name="s",
        num_cores=NC, num_subcores=NS)


@functools.lru_cache(maxsize=None)
def _make_sc_gather():
  return pl.kernel(
    _sc_gather_body,
    out_type=(jax.ShapeDtypeStruct((E, H), jnp.float32),
              jax.ShapeDtypeStruct((E, H), jnp.float32)),
    mesh=_get_mesh(),
    scratch_types=[
        pltpu.VMEM((NB, BG), jnp.int32),
        pltpu.VMEM((BG, H), jnp.float32),
        pltpu.SemaphoreType.DMA,
    ],
  )


def _sc_gather_body(x_hbm, idx2_hbm, xs_hbm, xd_hbm, idx_v, rows_v, sem):
    c = lax.axis_index("c")
    s = lax.axis_index("s")
    pltpu.sync_copy(idx2_hbm.at[c, s], idx_v)

    def run(out_hbm):
        def body(j, carry):
            pltpu.async_copy(x_hbm.at[idx_v.at[j]], rows_v, sem).wait()
            pltpu.sync_copy(rows_v, out_hbm.at[pl.ds(s * EPT + j * BG, BG)])
            return carry
        lax.fori_loop(0, NB, body, 0)

    @pl.when(c == 0)
    def _():
        run(xs_hbm)

    @pl.when(c == 1)
    def _():
        run(xd_hbm)


@functools.lru_cache(maxsize=None)
def _make_sc_scatter():
  return pl.kernel(
    _sc_scatter_body,
    out_type=(jax.ShapeDtypeStruct((NPAD, H), jnp.float32),
              jax.ShapeDtypeStruct((NPAD, H), jnp.float32)),
    mesh=_get_mesh(),
    scratch_types=[
        pltpu.VMEM((NB, BG), jnp.int32),
        pltpu.VMEM((BG, H), jnp.float32),
        pltpu.VMEM_SHARED((NPAD, H), jnp.float32),
        pltpu.SemaphoreType.DMA,
    ],
  )


def _sc_scatter_body(ys_hbm, yd_hbm, sidx2_hbm, zeros_hbm, mi_hbm, mo_hbm,
                     idx_v, rows_v, acc_sh, sem):
    c = lax.axis_index("c")
    s = lax.axis_index("s")
    pltpu.sync_copy(sidx2_hbm.at[c, s], idx_v)
    pltpu.sync_copy(zeros_hbm, acc_sh.at[pl.ds(s * ROWS_PT, ROWS_PT)])
    plsc.subcore_barrier()

    def run(y_hbm):
        def body(j, carry):
            pltpu.sync_copy(y_hbm.at[pl.ds(s * EPT + j * BG, BG)], rows_v)
            pltpu.sync_copy(rows_v, acc_sh.at[idx_v.at[j]], add=True)
            return carry
        lax.fori_loop(0, NB, body, 0)

    @pl.when(c == 0)
    def _():
        run(ys_hbm)

    @pl.when(c == 1)
    def _():
        run(yd_hbm)

    plsc.subcore_barrier()

    def out_copy(out_hbm):
        pltpu.sync_copy(acc_sh.at[pl.ds(s * ROWS_PT, ROWS_PT)],
                        out_hbm.at[pl.ds(s * ROWS_PT, ROWS_PT)])

    @pl.when(c == 0)
    def _():
        out_copy(mi_hbm)

    @pl.when(c == 1)
    def _():
        out_copy(mo_hbm)


TN = 2000
TE = 2560

_full = lambda shape: pl.BlockSpec(shape, lambda i: (0,) * len(shape))


def _input_body(x_ref, w_ref, b_ref, o_ref):
    o_ref[...] = jnp.tanh(jnp.dot(x_ref[...], w_ref[...]) + b_ref[...])


def _input_mlp(x, W_in, b_in):
    return pl.pallas_call(
        _input_body,
        grid=(N // TN,),
        in_specs=[pl.BlockSpec((TN, H), lambda i: (i, 0)),
                  _full((H, H)), _full((1, H))],
        out_specs=pl.BlockSpec((TN, H), lambda i: (i, 0)),
        out_shape=jax.ShapeDtypeStruct((N, H), jnp.float32),
    )(x, W_in, b_in.reshape(1, H))


def _edge_mlp_common(xs, xd, w1a, w1b, b1, w2, b2, w3, b3, w4r, b4):
    h = jnp.tanh(jnp.dot(xs, w1a) + jnp.dot(xd, w1b) + b1)
    h = jnp.tanh(jnp.dot(h, w2) + b2)
    h = jnp.tanh(jnp.dot(h, w3) + b3)
    logit = jnp.sum(h * w4r, axis=1, keepdims=True) + b4
    return jax.nn.sigmoid(logit)


def _edge_full_body(xs_ref, xd_ref, w1a, w1b, b1, w2, b2, w3, b3, w4r, b4,
                    e_ref, ys_ref, yd_ref):
    xs = xs_ref[...]
    xd = xd_ref[...]
    e = _edge_mlp_common(xs, xd, w1a[...], w1b[...], b1[...], w2[...],
                         b2[...], w3[...], b3[...], w4r[...], b4[...])
    e_ref[...] = e
    ys_ref[...] = e * xs
    yd_ref[...] = e * xd


def _edge_last_body(xs_ref, xd_ref, w1a, w1b, b1, w2, b2, w3, b3, w4r, b4,
                    e_ref):
    e = _edge_mlp_common(xs_ref[...], xd_ref[...], w1a[...], w1b[...],
                         b1[...], w2[...], b2[...], w3[...], b3[...],
                         w4r[...], b4[...])
    e_ref[...] = e


_EDGE_W_SPECS = [
    _full((H, H)), _full((H, H)), _full((1, H)),
    _full((H, H)), _full((1, H)),
    _full((H, H)), _full((1, H)),
    _full((1, H)), _full((1, 1)),
]


def _edge_full(xs, xd, ew):
    espec = pl.BlockSpec((TE, 1), lambda i: (i, 0))
    rspec = pl.BlockSpec((TE, H), lambda i: (i, 0))
    return pl.pallas_call(
        _edge_full_body,
        grid=(E // TE,),
        in_specs=[rspec, rspec] + _EDGE_W_SPECS,
        out_specs=(espec, rspec, rspec),
        out_shape=(jax.ShapeDtypeStruct((E, 1), jnp.float32),
                   jax.ShapeDtypeStruct((E, H), jnp.float32),
                   jax.ShapeDtypeStruct((E, H), jnp.float32)),
    )(xs, xd, *ew)


def _edge_last(xs, xd, ew):
    espec = pl.BlockSpec((TE, 1), lambda i: (i, 0))
    rspec = pl.BlockSpec((TE, H), lambda i: (i, 0))
    return pl.pallas_call(
        _edge_last_body,
        grid=(E // TE,),
        in_specs=[rspec, rspec] + _EDGE_W_SPECS,
        out_specs=espec,
        out_shape=jax.ShapeDtypeStruct((E, 1), jnp.float32),
    )(xs, xd, *ew)


def _node_body(mi_ref, mo_ref, x_ref, w1a, w1b, w1c, b1, w2, b2, w3, b3,
               w4, b4, o_ref):
    x = x_ref[...]
    g = jnp.tanh(jnp.dot(mi_ref[...], w1a[...]) + jnp.dot(mo_ref[...], w1b[...])
                 + jnp.dot(x, w1c[...]) + b1[...])
    g = jnp.tanh(jnp.dot(g, w2[...]) + b2[...])
    g = jnp.tanh(jnp.dot(g, w3[...]) + b3[...])
    g = jnp.tanh(jnp.dot(g, w4[...]) + b4[...])
    o_ref[...] = x + g


def _node_mlp(mi, mo, x, nw):
    rspec = pl.BlockSpec((TN, H), lambda i: (i, 0))
    wspecs = [_full((H, H)), _full((H, H)), _full((H, H)), _full((1, H)),
              _full((H, H)), _full((1, H)), _full((H, H)), _full((1, H)),
              _full((H, H)), _full((1, H))]
    return pl.pallas_call(
        _node_body,
        grid=(N // TN,),
        in_specs=[rspec, rspec, rspec] + wspecs,
        out_specs=rspec,
        out_shape=jax.ShapeDtypeStruct((N, H), jnp.float32),
    )(mi, mo, x, *nw)


def kernel(x, edge_index, W_in, b_in, eW1, eb1, eW2, eb2, eW3, eb3, eW4, eb4,
           nW1, nb1, nW2, nb2, nW3, nb3, nW4, nb4):
    src = edge_index[0].astype(jnp.int32)
    dst = edge_index[1].astype(jnp.int32)
    idx_g = jnp.stack([src.reshape(NS, NB, BG), dst.reshape(NS, NB, BG)])
    idx_s = jnp.stack([dst.reshape(NS, NB, BG), src.reshape(NS, NB, BG)])
    zeros = jnp.zeros((ROWS_PT, H), jnp.float32)

    ew = (eW1[:H], eW1[H:], eb1.reshape(1, H), eW2, eb2.reshape(1, H),
          eW3, eb3.reshape(1, H), eW4.reshape(1, H), eb4.reshape(1, 1))
    nw = (nW1[:H], nW1[H:2 * H], nW1[2 * H:], nb1.reshape(1, H),
          nW2, nb2.reshape(1, H), nW3, nb3.reshape(1, H),
          nW4, nb4.reshape(1, H))

    xcur = _input_mlp(x, W_in, b_in)
    e = None
    for n in range(4):
        xs, xd = _make_sc_gather()(xcur, idx_g)
        if n < 3:
            e, ys, yd = _edge_full(xs, xd, ew)
            mi, mo = _make_sc_scatter()(ys, yd, idx_s, zeros)
            xcur = _node_mlp(mi, mo, xcur, nw)
        else:
            e = _edge_last(xs, xd, ew)
    return e.reshape(E)


# speedup vs baseline: 2.3540x; 1.0238x over previous
import functools

import jax
import jax.numpy as jnp
from jax import lax
from jax.experimental import pallas as pl
from jax.experimental.pallas import tpu as pltpu
from jax.experimental.pallas import tpu_sc as plsc

N = 10000
E = 320000
H = 128

NC = 2
NS = 16
EPT = E // NS
NPAD = 10240
ROWS_PT = NPAD // NS

BG = 40
NB = EPT // BG
KG = 5
ITG = NB // (2 * KG)

BS = 40
NBS = EPT // BS
KS = 1
ITS = NBS // (2 * KS)


@functools.lru_cache(maxsize=None)
def _get_mesh():
    return plsc.VectorSubcoreMesh(
        core_axis_name="c", subcore_axis_name="s",
        num_cores=NC, num_subcores=NS)


@functools.lru_cache(maxsize=None)
def _make_sc_gather():
  return pl.kernel(
    _sc_gather_body,
    out_type=(jax.ShapeDtypeStruct((E, H), jnp.float32),
              jax.ShapeDtypeStruct((E, H), jnp.float32)),
    mesh=_get_mesh(),
    scratch_types=[
        pltpu.VMEM((NB, BG), jnp.int32),
        pltpu.VMEM((2 * KG, BG, H), jnp.float32),
        pltpu.SemaphoreType.DMA,
        pltpu.SemaphoreType.DMA,
        pltpu.SemaphoreType.DMA,
        pltpu.SemaphoreType.DMA,
    ],
  )


def _sc_gather_body(x_hbm, idx2_hbm, xs_hbm, xd_hbm, idx_v, rows_v,
                    gs0, gs1, ws0, ws1):
    c = lax.axis_index("c")
    s = lax.axis_index("s")
    pltpu.sync_copy(idx2_hbm.at[c, s], idx_v)
    base = s * EPT
    gsem = (gs0, gs1)
    wsem = (ws0, ws1)

    def run(out_hbm):
        def round_(rr, bank):
            j0 = (2 * rr + bank) * KG

            @pl.when(rr > 0)
            def _():
                for b in range(KG):
                    pltpu.make_async_copy(
                        x_hbm.at[pl.ds(0, BG)], rows_v.at[bank * KG + b],
                        wsem[bank]).wait()

            descs = [
                pltpu.async_copy(x_hbm.at[idx_v.at[j0 + b]],
                                 rows_v.at[bank * KG + b], gsem[bank])
                for b in range(KG)
            ]
            for d in descs:
                d.wait()
            for b in range(KG):
                pltpu.async_copy(
                    rows_v.at[bank * KG + b],
                    out_hbm.at[pl.ds(base + (j0 + b) * BG, BG)],
                    wsem[bank])

        def body(rr, carry):
            round_(rr, 0)
            round_(rr, 1)
            return carry

        lax.fori_loop(0, ITG, body, 0)
        for bank in range(2):
            for b in range(KG):
                pltpu.make_async_copy(
                    x_hbm.at[pl.ds(0, BG)], rows_v.at[bank * KG + b],
                    wsem[bank]).wait()

    @pl.when(c == 0)
    def _():
        run(xs_hbm)

    @pl.when(c == 1)
    def _():
        run(xd_hbm)


@functools.lru_cache(maxsize=None)
def _make_sc_scatter():
  return pl.kernel(
    _sc_scatter_body,
    out_type=(jax.ShapeDtypeStruct((NPAD, H), jnp.float32),
              jax.ShapeDtypeStruct((NPAD, H), jnp.float32)),
    mesh=_get_mesh(),
    scratch_types=[
        pltpu.VMEM((EPT + 16,), jnp.int32),
        pltpu.VMEM((2, BS), jnp.int32),
        pltpu.VMEM((2, BS, H), jnp.float32),
        pltpu.VMEM_SHARED((NPAD, H), jnp.float32),
        pltpu.SemaphoreType.DMA,
        pltpu.SemaphoreType.DMA,
        pltpu.SemaphoreType.DMA,
        pltpu.SemaphoreType.DMA,
    ],
  )


def _sc_scatter_body(xs_hbm, xd_hbm, comb2_hbm, zeros_hbm, mi_hbm, mo_hbm,
                     comb_v, idx_st, rows_v, acc_sh,
                     gs0, gs1, ws0, ws1):
    c = lax.axis_index("c")
    s = lax.axis_index("s")
    pltpu.sync_copy(comb2_hbm.at[c, s], comb_v)
    pltpu.sync_copy(zeros_hbm, acc_sh.at[pl.ds(s * ROWS_PT, ROWS_PT)])
    plsc.subcore_barrier()
    base = s * EPT
    gsem = (gs0, gs1)
    wsem = (ws0, ws1)

    def run(y_hbm):
        def round_(rr, bank):
            r = 2 * rr + bank

            @pl.when(rr > 0)
            def _():
                pltpu.make_async_copy(
                    y_hbm.at[pl.ds(0, BS)], rows_v.at[bank],
                    wsem[bank]).wait()

            d = pltpu.async_copy(
                y_hbm.at[pl.ds(base + r * BS, BS)], rows_v.at[bank],
                gsem[bank])
            for o in (0, 16, BS - 16):
                cv = comb_v[pl.ds(r * BS + o, 16)]
                idx_st[bank, pl.ds(o, 16)] = cv & jnp.int32(0x3FFF)
            d.wait()
            def sbody(i, carry):
                cv = comb_v[pl.ds(r * BS + i, 16)]
                ei = ((cv[0] >> 14).astype(jnp.float32)
                      * jnp.float32(1.0 / 131071.0))
                esp = jnp.full((16,), ei, jnp.float32)
                for k in range(8):
                    sl = pl.ds(k * 16, 16)
                    rows_v[bank, i, sl] = rows_v[bank, i, sl] * esp
                return carry

            lax.fori_loop(0, BS, sbody, 0)
            pltpu.async_copy(rows_v.at[bank], acc_sh.at[idx_st.at[bank]],
                             wsem[bank], add=True)

        def body(rr, carry):
            round_(rr, 0)
            round_(rr, 1)
            return carry

        lax.fori_loop(0, ITS, body, 0)
        for bank in range(2):
            pltpu.make_async_copy(
                y_hbm.at[pl.ds(0, BS)], rows_v.at[bank], wsem[bank]).wait()

    @pl.when(c == 0)
    def _():
        run(xs_hbm)

    @pl.when(c == 1)
    def _():
        run(xd_hbm)

    plsc.subcore_barrier()

    def out_copy(out_hbm):
        pltpu.sync_copy(acc_sh.at[pl.ds(s * ROWS_PT, ROWS_PT)],
                        out_hbm.at[pl.ds(s * ROWS_PT, ROWS_PT)])

    @pl.when(c == 0)
    def _():
        out_copy(mi_hbm)

    @pl.when(c == 1)
    def _():
        out_copy(mo_hbm)


TN = 2000
TE = 2560

_full = lambda shape: pl.BlockSpec(shape, lambda i: (0,) * len(shape))


def _input_body(x_ref, w_ref, b_ref, o_ref):
    o_ref[...] = jnp.tanh(jnp.dot(x_ref[...], w_ref[...]) + b_ref[...])


def _input_mlp(x, W_in, b_in):
    return pl.pallas_call(
        _input_body,
        grid=(N // TN,),
        in_specs=[pl.BlockSpec((TN, H), lambda i: (i, 0)),
                  _full((H, H)), _full((1, H))],
        out_specs=pl.BlockSpec((TN, H), lambda i: (i, 0)),
        out_shape=jax.ShapeDtypeStruct((N, H), jnp.float32),
    )(x, W_in, b_in.reshape(1, H))


def _edge_body(xs_ref, xd_ref, w1a, w1b, b1, w2, b2, w3, b3, w4r, b4,
               e_ref):
    h = jnp.tanh(jnp.dot(xs_ref[...], w1a[...])
                 + jnp.dot(xd_ref[...], w1b[...]) + b1[...])
    h = jnp.tanh(jnp.dot(h, w2[...]) + b2[...])
    h = jnp.tanh(jnp.dot(h, w3[...]) + b3[...])
    logit = jnp.sum(h * w4r[...], axis=1, keepdims=True) + b4[...]
    e_ref[...] = jax.nn.sigmoid(logit)


_EDGE_W_SPECS = [
    _full((H, H)), _full((H, H)), _full((1, H)),
    _full((H, H)), _full((1, H)),
    _full((H, H)), _full((1, H)),
    _full((1, H)), _full((1, 1)),
]


def _edge_mlp(xs, xd, ew):
    espec = pl.BlockSpec((TE, 1), lambda i: (i, 0))
    rspec = pl.BlockSpec((TE, H), lambda i: (i, 0))
    return pl.pallas_call(
        _edge_body,
        grid=(E // TE,),
        in_specs=[rspec, rspec] + _EDGE_W_SPECS,
        out_specs=espec,
        out_shape=jax.ShapeDtypeStruct((E, 1), jnp.float32),
    )(xs, xd, *ew)


def _node_body(mi_ref, mo_ref, x_ref, w1a, w1b, w1c, b1, w2, b2, w3, b3,
               w4, b4, o_ref):
    x = x_ref[...]
    g = jnp.tanh(jnp.dot(mi_ref[...], w1a[...]) + jnp.dot(mo_ref[...], w1b[...])
                 + jnp.dot(x, w1c[...]) + b1[...])
    g = jnp.tanh(jnp.dot(g, w2[...]) + b2[...])
    g = jnp.tanh(jnp.dot(g, w3[...]) + b3[...])
    g = jnp.tanh(jnp.dot(g, w4[...]) + b4[...])
    o_ref[...] = x + g


def _node_mlp(mi, mo, x, nw):
    rspec = pl.BlockSpec((TN, H), lambda i: (i, 0))
    wspecs = [_full((H, H)), _full((H, H)), _full((H, H)), _full((1, H)),
              _full((H, H)), _full((1, H)), _full((H, H)), _full((1, H)),
              _full((H, H)), _full((1, H))]
    return pl.pallas_call(
        _node_body,
        grid=(N // TN,),
        in_specs=[rspec, rspec, rspec] + wspecs,
        out_specs=rspec,
        out_shape=jax.ShapeDtypeStruct((N, H), jnp.float32),
    )(mi, mo, x, *nw)


def kernel(x, edge_index, W_in, b_in, eW1, eb1, eW2, eb2, eW3, eb3, eW4, eb4,
           nW1, nb1, nW2, nb2, nW3, nb3, nW4, nb4):
    src = edge_index[0].astype(jnp.int32)
    dst = edge_index[1].astype(jnp.int32)
    idx_g = jnp.stack([src.reshape(NS, NB, BG), dst.reshape(NS, NB, BG)])
    idx_s2 = jnp.stack([dst, src])
    zeros = jnp.zeros((ROWS_PT, H), jnp.float32)

    ew = (eW1[:H], eW1[H:], eb1.reshape(1, H), eW2, eb2.reshape(1, H),
          eW3, eb3.reshape(1, H), eW4.reshape(1, H), eb4.reshape(1, 1))
    nw = (nW1[:H], nW1[H:2 * H], nW1[2 * H:], nb1.reshape(1, H),
          nW2, nb2.reshape(1, H), nW3, nb3.reshape(1, H),
          nW4, nb4.reshape(1, H))

    xcur = _input_mlp(x, W_in, b_in)
    e = None
    for n in range(4):
        xs, xd = _make_sc_gather()(xcur, idx_g)
        e = _edge_mlp(xs, xd, ew)
        if n < 3:
            eint = jnp.round(e.reshape(E) * 131071.0).astype(jnp.int32)
            comb2 = jnp.pad((idx_s2 | (eint << 14)).reshape(2, NS, EPT),
                            ((0, 0), (0, 0), (0, 16)))
            mi, mo = _make_sc_scatter()(xs, xd, comb2, zeros)
            xcur = _node_mlp(mi, mo, xcur, nw)
    return e.reshape(E)


# speedup vs baseline: 2.9127x; 1.2374x over previous
import functools

import jax
import jax.numpy as jnp
from jax import lax
from jax.experimental import pallas as pl
from jax.experimental.pallas import tpu as pltpu
from jax.experimental.pallas import tpu_sc as plsc

N = 10000
E = 320000
H = 128

NC = 2
NS = 16
EPT = E // NS
NPAD = 10240
ROWS_PT = NPAD // NS

BG = 40
NB = EPT // BG
KG = 5
ITG = NB // (2 * KG)

BS = 40
NBS = EPT // BS
KS = 1
ITS = NBS // (2 * KS)


@functools.lru_cache(maxsize=None)
def _get_mesh():
    return plsc.VectorSubcoreMesh(
        core_axis_name="c", subcore_axis_name="s",
        num_cores=NC, num_subcores=NS)


@functools.lru_cache(maxsize=None)
def _make_sc_gather():
  return pl.kernel(
    _sc_gather_body,
    out_type=(jax.ShapeDtypeStruct((E, H), jnp.float32),
              jax.ShapeDtypeStruct((E, H), jnp.float32)),
    mesh=_get_mesh(),
    scratch_types=[
        pltpu.VMEM((NB, BG), jnp.int32),
        pltpu.VMEM((2 * KG, BG, H), jnp.float32),
        pltpu.SemaphoreType.DMA,
        pltpu.SemaphoreType.DMA,
        pltpu.SemaphoreType.DMA,
        pltpu.SemaphoreType.DMA,
    ],
  )


def _sc_gather_body(x_hbm, idx2_hbm, xs_hbm, xd_hbm, idx_v, rows_v,
                    gs0, gs1, ws0, ws1):
    c = lax.axis_index("c")
    s = lax.axis_index("s")
    pltpu.sync_copy(idx2_hbm.at[c, s], idx_v)
    base = s * EPT
    gsem = (gs0, gs1)
    wsem = (ws0, ws1)

    def run(out_hbm):
        def round_(rr, bank):
            j0 = (2 * rr + bank) * KG

            @pl.when(rr > 0)
            def _():
                for b in range(KG):
                    pltpu.make_async_copy(
                        x_hbm.at[pl.ds(0, BG)], rows_v.at[bank * KG + b],
                        wsem[bank]).wait()

            descs = [
                pltpu.async_copy(x_hbm.at[idx_v.at[j0 + b]],
                                 rows_v.at[bank * KG + b], gsem[bank])
                for b in range(KG)
            ]
            for d in descs:
                d.wait()
            for b in range(KG):
                pltpu.async_copy(
                    rows_v.at[bank * KG + b],
                    out_hbm.at[pl.ds(base + (j0 + b) * BG, BG)],
                    wsem[bank])

        def body(rr, carry):
            round_(rr, 0)
            round_(rr, 1)
            return carry

        lax.fori_loop(0, ITG, body, 0)
        for bank in range(2):
            for b in range(KG):
                pltpu.make_async_copy(
                    x_hbm.at[pl.ds(0, BG)], rows_v.at[bank * KG + b],
                    wsem[bank]).wait()

    @pl.when(c == 0)
    def _():
        run(xs_hbm)

    @pl.when(c == 1)
    def _():
        run(xd_hbm)


@functools.lru_cache(maxsize=None)
def _make_sc_scatter():
  return pl.kernel(
    _sc_scatter_body,
    out_type=(jax.ShapeDtypeStruct((NPAD, H), jnp.float32),
              jax.ShapeDtypeStruct((NPAD, H), jnp.float32)),
    mesh=_get_mesh(),
    scratch_types=[
        pltpu.VMEM((EPT + 16,), jnp.int32),
        pltpu.VMEM((2, BS), jnp.int32),
        pltpu.VMEM((2, BS, H), jnp.float32),
        pltpu.VMEM_SHARED((NPAD, H), jnp.float32),
        pltpu.SemaphoreType.DMA,
        pltpu.SemaphoreType.DMA,
        pltpu.SemaphoreType.DMA,
        pltpu.SemaphoreType.DMA,
    ],
  )


def _sc_scatter_body(xs_hbm, xd_hbm, comb2_hbm, zeros_hbm, mi_hbm, mo_hbm,
                     comb_v, idx_st, rows_v, acc_sh,
                     gs0, gs1, ws0, ws1):
    c = lax.axis_index("c")
    s = lax.axis_index("s")
    pltpu.sync_copy(comb2_hbm.at[c, s], comb_v)
    pltpu.sync_copy(zeros_hbm, acc_sh.at[pl.ds(s * ROWS_PT, ROWS_PT)])
    plsc.subcore_barrier()
    base = s * EPT
    gsem = (gs0, gs1)
    wsem = (ws0, ws1)

    def run(y_hbm):
        def round_(rr, bank):
            r = 2 * rr + bank

            @pl.when(rr > 0)
            def _():
                pltpu.make_async_copy(
                    y_hbm.at[pl.ds(0, BS)], rows_v.at[bank],
                    wsem[bank]).wait()

            d = pltpu.async_copy(
                y_hbm.at[pl.ds(base + r * BS, BS)], rows_v.at[bank],
                gsem[bank])
            for o in (0, 16, BS - 16):
                cv = comb_v[pl.ds(r * BS + o, 16)]
                idx_st[bank, pl.ds(o, 16)] = cv & jnp.int32(0x3FFF)
            d.wait()
            for o, lanes in ((0, range(0, 16)), (16, range(0, 16)),
                             (BS - 16, range(2 * 16 - (BS - 16), 16))):
                ev = comb_v[pl.ds(r * BS + o, 16)]
                evf = ((ev >> 14).astype(jnp.float32)
                       * jnp.float32(1.0 / 131071.0))
                for l in lanes:
                    esp = evf.at[jnp.full((16,), l, jnp.int32)].get(
                        mode="promise_in_bounds")
                    i = o + l
                    for k in range(8):
                        sl = pl.ds(k * 16, 16)
                        rows_v[bank, i, sl] = rows_v[bank, i, sl] * esp
            pltpu.async_copy(rows_v.at[bank], acc_sh.at[idx_st.at[bank]],
                             wsem[bank], add=True)

        def body(rr, carry):
            round_(rr, 0)
            round_(rr, 1)
            return carry

        lax.fori_loop(0, ITS, body, 0)
        for bank in range(2):
            pltpu.make_async_copy(
                y_hbm.at[pl.ds(0, BS)], rows_v.at[bank], wsem[bank]).wait()

    @pl.when(c == 0)
    def _():
        run(xs_hbm)

    @pl.when(c == 1)
    def _():
        run(xd_hbm)

    plsc.subcore_barrier()

    def out_copy(out_hbm):
        pltpu.sync_copy(acc_sh.at[pl.ds(s * ROWS_PT, ROWS_PT)],
                        out_hbm.at[pl.ds(s * ROWS_PT, ROWS_PT)])

    @pl.when(c == 0)
    def _():
        out_copy(mi_hbm)

    @pl.when(c == 1)
    def _():
        out_copy(mo_hbm)


TN = 2000
TE = 2560

_full = lambda shape: pl.BlockSpec(shape, lambda i: (0,) * len(shape))


def _input_body(x_ref, w_ref, b_ref, o_ref):
    o_ref[...] = jnp.tanh(jnp.dot(x_ref[...], w_ref[...]) + b_ref[...])


def _input_mlp(x, W_in, b_in):
    return pl.pallas_call(
        _input_body,
        grid=(N // TN,),
        in_specs=[pl.BlockSpec((TN, H), lambda i: (i, 0)),
                  _full((H, H)), _full((1, H))],
        out_specs=pl.BlockSpec((TN, H), lambda i: (i, 0)),
        out_shape=jax.ShapeDtypeStruct((N, H), jnp.float32),
    )(x, W_in, b_in.reshape(1, H))


def _edge_body(xs_ref, xd_ref, w1a, w1b, b1, w2, b2, w3, b3, w4r, b4,
               e_ref):
    h = jnp.tanh(jnp.dot(xs_ref[...], w1a[...])
                 + jnp.dot(xd_ref[...], w1b[...]) + b1[...])
    h = jnp.tanh(jnp.dot(h, w2[...]) + b2[...])
    h = jnp.tanh(jnp.dot(h, w3[...]) + b3[...])
    logit = jnp.sum(h * w4r[...], axis=1, keepdims=True) + b4[...]
    e_ref[...] = jax.nn.sigmoid(logit)


_EDGE_W_SPECS = [
    _full((H, H)), _full((H, H)), _full((1, H)),
    _full((H, H)), _full((1, H)),
    _full((H, H)), _full((1, H)),
    _full((1, H)), _full((1, 1)),
]


def _edge_mlp(xs, xd, ew):
    espec = pl.BlockSpec((TE, 1), lambda i: (i, 0))
    rspec = pl.BlockSpec((TE, H), lambda i: (i, 0))
    return pl.pallas_call(
        _edge_body,
        grid=(E // TE,),
        in_specs=[rspec, rspec] + _EDGE_W_SPECS,
        out_specs=espec,
        out_shape=jax.ShapeDtypeStruct((E, 1), jnp.float32),
    )(xs, xd, *ew)


def _node_body(mi_ref, mo_ref, x_ref, w1a, w1b, w1c, b1, w2, b2, w3, b3,
               w4, b4, o_ref):
    x = x_ref[...]
    g = jnp.tanh(jnp.dot(mi_ref[...], w1a[...]) + jnp.dot(mo_ref[...], w1b[...])
                 + jnp.dot(x, w1c[...]) + b1[...])
    g = jnp.tanh(jnp.dot(g, w2[...]) + b2[...])
    g = jnp.tanh(jnp.dot(g, w3[...]) + b3[...])
    g = jnp.tanh(jnp.dot(g, w4[...]) + b4[...])
    o_ref[...] = x + g


def _node_mlp(mi, mo, x, nw):
    rspec = pl.BlockSpec((TN, H), lambda i: (i, 0))
    wspecs = [_full((H, H)), _full((H, H)), _full((H, H)), _full((1, H)),
              _full((H, H)), _full((1, H)), _full((H, H)), _full((1, H)),
              _full((H, H)), _full((1, H))]
    return pl.pallas_call(
        _node_body,
        grid=(N // TN,),
        in_specs=[rspec, rspec, rspec] + wspecs,
        out_specs=rspec,
        out_shape=jax.ShapeDtypeStruct((N, H), jnp.float32),
    )(mi, mo, x, *nw)


def kernel(x, edge_index, W_in, b_in, eW1, eb1, eW2, eb2, eW3, eb3, eW4, eb4,
           nW1, nb1, nW2, nb2, nW3, nb3, nW4, nb4):
    src = edge_index[0].astype(jnp.int32)
    dst = edge_index[1].astype(jnp.int32)
    idx_g = jnp.stack([src.reshape(NS, NB, BG), dst.reshape(NS, NB, BG)])
    idx_s2 = jnp.stack([dst, src])
    zeros = jnp.zeros((ROWS_PT, H), jnp.float32)

    ew = (eW1[:H], eW1[H:], eb1.reshape(1, H), eW2, eb2.reshape(1, H),
          eW3, eb3.reshape(1, H), eW4.reshape(1, H), eb4.reshape(1, 1))
    nw = (nW1[:H], nW1[H:2 * H], nW1[2 * H:], nb1.reshape(1, H),
          nW2, nb2.reshape(1, H), nW3, nb3.reshape(1, H),
          nW4, nb4.reshape(1, H))

    xcur = _input_mlp(x, W_in, b_in)
    e = None
    for n in range(4):
        xs, xd = _make_sc_gather()(xcur, idx_g)
        e = _edge_mlp(xs, xd, ew)
        if n < 3:
            eint = jnp.round(e.reshape(E) * 131071.0).astype(jnp.int32)
            comb2 = jnp.pad((idx_s2 | (eint << 14)).reshape(2, NS, EPT),
                            ((0, 0), (0, 0), (0, 16)))
            mi, mo = _make_sc_scatter()(xs, xd, comb2, zeros)
            xcur = _node_mlp(mi, mo, xcur, nw)
    return e.reshape(E)


# speedup vs baseline: 3.8408x; 1.3186x over previous
import functools

import jax
import jax.numpy as jnp
from jax import lax
from jax.experimental import pallas as pl
from jax.experimental.pallas import tpu as pltpu
from jax.experimental.pallas import tpu_sc as plsc

N = 10000
E = 320000
H = 128

NC = 2
NS = 16
EPT = E // NS
NPAD = 10240
ROWS_PT = NPAD // NS

BG = 40
NB = EPT // BG
KG = 5
ITG = NB // (2 * KG)

BS = 40
NBS = EPT // BS
NBANKS = 4
ITS = NBS // NBANKS


@functools.lru_cache(maxsize=None)
def _get_mesh():
    return plsc.VectorSubcoreMesh(
        core_axis_name="c", subcore_axis_name="s",
        num_cores=NC, num_subcores=NS)


@functools.lru_cache(maxsize=None)
def _make_sc_gather():
  return pl.kernel(
    _sc_gather_body,
    out_type=(jax.ShapeDtypeStruct((E, H), jnp.float32),
              jax.ShapeDtypeStruct((E, H), jnp.float32)),
    mesh=_get_mesh(),
    scratch_types=[
        pltpu.VMEM((NB, BG), jnp.int32),
        pltpu.VMEM((2 * KG, BG, H), jnp.float32),
        pltpu.SemaphoreType.DMA,
        pltpu.SemaphoreType.DMA,
        pltpu.SemaphoreType.DMA,
        pltpu.SemaphoreType.DMA,
    ],
  )


def _sc_gather_body(x_hbm, idx2_hbm, xs_hbm, xd_hbm, idx_v, rows_v,
                    gs0, gs1, ws0, ws1):
    c = lax.axis_index("c")
    s = lax.axis_index("s")
    pltpu.sync_copy(idx2_hbm.at[c, s], idx_v)
    base = s * EPT
    gsem = (gs0, gs1)
    wsem = (ws0, ws1)

    def run(out_hbm):
        def round_(rr, bank):
            j0 = (2 * rr + bank) * KG

            @pl.when(rr > 0)
            def _():
                for b in range(KG):
                    pltpu.make_async_copy(
                        x_hbm.at[pl.ds(0, BG)], rows_v.at[bank * KG + b],
                        wsem[bank]).wait()

            descs = [
                pltpu.async_copy(x_hbm.at[idx_v.at[j0 + b]],
                                 rows_v.at[bank * KG + b], gsem[bank])
                for b in range(KG)
            ]
            for d in descs:
                d.wait()
            for b in range(KG):
                pltpu.async_copy(
                    rows_v.at[bank * KG + b],
                    out_hbm.at[pl.ds(base + (j0 + b) * BG, BG)],
                    wsem[bank])

        def body(rr, carry):
            round_(rr, 0)
            round_(rr, 1)
            return carry

        lax.fori_loop(0, ITG, body, 0)
        for bank in range(2):
            for b in range(KG):
                pltpu.make_async_copy(
                    x_hbm.at[pl.ds(0, BG)], rows_v.at[bank * KG + b],
                    wsem[bank]).wait()

    @pl.when(c == 0)
    def _():
        run(xs_hbm)

    @pl.when(c == 1)
    def _():
        run(xd_hbm)


@functools.lru_cache(maxsize=None)
def _make_sc_scatter():
  return pl.kernel(
    _sc_scatter_body,
    out_type=(jax.ShapeDtypeStruct((NPAD, H), jnp.float32),
              jax.ShapeDtypeStruct((NPAD, H), jnp.float32)),
    mesh=_get_mesh(),
    scratch_types=[
        pltpu.VMEM((EPT + 16,), jnp.int32),
        pltpu.VMEM((NBANKS, BS), jnp.int32),
        pltpu.VMEM((NBANKS, BS, H), jnp.float32),
        pltpu.VMEM_SHARED((NPAD, H), jnp.float32),
    ] + [pltpu.SemaphoreType.DMA] * (2 * NBANKS),
  )


def _sc_scatter_body(xs_hbm, xd_hbm, comb2_hbm, zeros_hbm, mi_hbm, mo_hbm,
                     comb_v, idx_st, rows_v, acc_sh,
                     gs0, gs1, gs2, gs3, ws0, ws1, ws2, ws3):
    c = lax.axis_index("c")
    s = lax.axis_index("s")
    pltpu.sync_copy(comb2_hbm.at[c, s], comb_v)
    pltpu.sync_copy(zeros_hbm, acc_sh.at[pl.ds(s * ROWS_PT, ROWS_PT)])
    plsc.subcore_barrier()
    base = s * EPT
    gsem = (gs0, gs1, gs2, gs3)
    wsem = (ws0, ws1, ws2, ws3)

    def run(y_hbm):
        pltpu.async_copy(y_hbm.at[pl.ds(base, BS)], rows_v.at[0], gsem[0])
        pltpu.async_copy(y_hbm.at[pl.ds(base + BS, BS)], rows_v.at[1],
                         gsem[1])

        def slot(rr, bank):
            r = NBANKS * rr + bank
            nb = (bank + 2) % NBANKS

            def drain_w():
                pltpu.make_async_copy(
                    y_hbm.at[pl.ds(0, BS)], rows_v.at[nb], wsem[nb]).wait()

            def fire_g():
                pltpu.async_copy(
                    y_hbm.at[pl.ds(base + (r + 2) * BS, BS)],
                    rows_v.at[nb], gsem[nb])

            if bank < 2:
                @pl.when(rr > 0)
                def _():
                    drain_w()
                fire_g()
            else:
                drain_w()

                @pl.when(rr < ITS - 1)
                def _():
                    fire_g()
            for o in (0, 16, BS - 16):
                cv = comb_v[pl.ds(r * BS + o, 16)]
                idx_st[bank, pl.ds(o, 16)] = cv & jnp.int32(0x3FFF)
            pltpu.make_async_copy(
                y_hbm.at[pl.ds(0, BS)], rows_v.at[bank], gsem[bank]).wait()
            for o, lanes in ((0, range(0, 16)), (16, range(0, 16)),
                             (BS - 16, range(2 * 16 - (BS - 16), 16))):
                ev = comb_v[pl.ds(r * BS + o, 16)]
                evf = ((ev >> 14).astype(jnp.float32)
                       * jnp.float32(1.0 / 131071.0))
                for l in lanes:
                    esp = evf.at[jnp.full((16,), l, jnp.int32)].get(
                        mode="promise_in_bounds")
                    i = o + l
                    for k in range(8):
                        sl = pl.ds(k * 16, 16)
                        rows_v[bank, i, sl] = rows_v[bank, i, sl] * esp
            pltpu.async_copy(rows_v.at[bank], acc_sh.at[idx_st.at[bank]],
                             wsem[bank], add=True)

        def body(rr, carry):
            for bank in range(NBANKS):
                slot(rr, bank)
            return carry

        lax.fori_loop(0, ITS, body, 0)
        for bank in (2, 3):
            pltpu.make_async_copy(
                y_hbm.at[pl.ds(0, BS)], rows_v.at[bank], wsem[bank]).wait()

    @pl.when(c == 0)
    def _():
        run(xs_hbm)

    @pl.when(c == 1)
    def _():
        run(xd_hbm)

    plsc.subcore_barrier()

    def out_copy(out_hbm):
        pltpu.sync_copy(acc_sh.at[pl.ds(s * ROWS_PT, ROWS_PT)],
                        out_hbm.at[pl.ds(s * ROWS_PT, ROWS_PT)])

    @pl.when(c == 0)
    def _():
        out_copy(mi_hbm)

    @pl.when(c == 1)
    def _():
        out_copy(mo_hbm)


TN = 2000
TE = 2560

_full = lambda shape: pl.BlockSpec(shape, lambda i: (0,) * len(shape))


def _input_body(x_ref, w_ref, b_ref, o_ref):
    o_ref[...] = jnp.tanh(jnp.dot(x_ref[...], w_ref[...]) + b_ref[...])


def _input_mlp(x, W_in, b_in):
    return pl.pallas_call(
        _input_body,
        grid=(N // TN,),
        in_specs=[pl.BlockSpec((TN, H), lambda i: (i, 0)),
                  _full((H, H)), _full((1, H))],
        out_specs=pl.BlockSpec((TN, H), lambda i: (i, 0)),
        out_shape=jax.ShapeDtypeStruct((N, H), jnp.float32),
    )(x, W_in, b_in.reshape(1, H))


def _edge_body(xs_ref, xd_ref, w1a, w1b, b1, w2, b2, w3, b3, w4r, b4,
               e_ref):
    h = jnp.tanh(jnp.dot(xs_ref[...], w1a[...])
                 + jnp.dot(xd_ref[...], w1b[...]) + b1[...])
    h = jnp.tanh(jnp.dot(h, w2[...]) + b2[...])
    h = jnp.tanh(jnp.dot(h, w3[...]) + b3[...])
    logit = jnp.sum(h * w4r[...], axis=1, keepdims=True) + b4[...]
    e_ref[...] = jax.nn.sigmoid(logit)


_EDGE_W_SPECS = [
    _full((H, H)), _full((H, H)), _full((1, H)),
    _full((H, H)), _full((1, H)),
    _full((H, H)), _full((1, H)),
    _full((1, H)), _full((1, 1)),
]


def _edge_mlp(xs, xd, ew):
    espec = pl.BlockSpec((TE, 1), lambda i: (i, 0))
    rspec = pl.BlockSpec((TE, H), lambda i: (i, 0))
    return pl.pallas_call(
        _edge_body,
        grid=(E // TE,),
        in_specs=[rspec, rspec] + _EDGE_W_SPECS,
        out_specs=espec,
        out_shape=jax.ShapeDtypeStruct((E, 1), jnp.float32),
    )(xs, xd, *ew)


def _node_body(mi_ref, mo_ref, x_ref, w1a, w1b, w1c, b1, w2, b2, w3, b3,
               w4, b4, o_ref):
    x = x_ref[...]
    g = jnp.tanh(jnp.dot(mi_ref[...], w1a[...]) + jnp.dot(mo_ref[...], w1b[...])
                 + jnp.dot(x, w1c[...]) + b1[...])
    g = jnp.tanh(jnp.dot(g, w2[...]) + b2[...])
    g = jnp.tanh(jnp.dot(g, w3[...]) + b3[...])
    g = jnp.tanh(jnp.dot(g, w4[...]) + b4[...])
    o_ref[...] = x + g


def _node_mlp(mi, mo, x, nw):
    rspec = pl.BlockSpec((TN, H), lambda i: (i, 0))
    wspecs = [_full((H, H)), _full((H, H)), _full((H, H)), _full((1, H)),
              _full((H, H)), _full((1, H)), _full((H, H)), _full((1, H)),
              _full((H, H)), _full((1, H))]
    return pl.pallas_call(
        _node_body,
        grid=(N // TN,),
        in_specs=[rspec, rspec, rspec] + wspecs,
        out_specs=rspec,
        out_shape=jax.ShapeDtypeStruct((N, H), jnp.float32),
    )(mi, mo, x, *nw)


def kernel(x, edge_index, W_in, b_in, eW1, eb1, eW2, eb2, eW3, eb3, eW4, eb4,
           nW1, nb1, nW2, nb2, nW3, nb3, nW4, nb4):
    src = edge_index[0].astype(jnp.int32)
    dst = edge_index[1].astype(jnp.int32)
    idx_g = jnp.stack([src.reshape(NS, NB, BG), dst.reshape(NS, NB, BG)])
    idx_s2 = jnp.stack([dst, src])
    zeros = jnp.zeros((ROWS_PT, H), jnp.float32)

    ew = (eW1[:H], eW1[H:], eb1.reshape(1, H), eW2, eb2.reshape(1, H),
          eW3, eb3.reshape(1, H), eW4.reshape(1, H), eb4.reshape(1, 1))
    nw = (nW1[:H], nW1[H:2 * H], nW1[2 * H:], nb1.reshape(1, H),
          nW2, nb2.reshape(1, H), nW3, nb3.reshape(1, H),
          nW4, nb4.reshape(1, H))

    xcur = _input_mlp(x, W_in, b_in)
    e = None
    for n in range(4):
        xs, xd = _make_sc_gather()(xcur, idx_g)
        e = _edge_mlp(xs, xd, ew)
        if n < 3:
            eint = jnp.round(e.reshape(E) * 131071.0).astype(jnp.int32)
            comb2 = jnp.pad((idx_s2 | (eint << 14)).reshape(2, NS, EPT),
                            ((0, 0), (0, 0), (0, 16)))
            mi, mo = _make_sc_scatter()(xs, xd, comb2, zeros)
            xcur = _node_mlp(mi, mo, xcur, nw)
    return e.reshape(E)


# speedup vs baseline: 3.9603x; 1.0311x over previous
import functools

import jax
import jax.numpy as jnp
from jax import lax
from jax.experimental import pallas as pl
from jax.experimental.pallas import tpu as pltpu
from jax.experimental.pallas import tpu_sc as plsc

N = 10000
E = 320000
H = 128

NC = 2
NS = 16
EPT = E // NS
NPAD = 10240
ROWS_PT = NPAD // NS

BG = 80
NB = EPT // BG
KG = 5
ITG = NB // (2 * KG)

BS = 40
NBS = EPT // BS
NBANKS = 4
ITS = NBS // NBANKS


@functools.lru_cache(maxsize=None)
def _get_mesh():
    return plsc.VectorSubcoreMesh(
        core_axis_name="c", subcore_axis_name="s",
        num_cores=NC, num_subcores=NS)


@functools.lru_cache(maxsize=None)
def _make_sc_gather():
  return pl.kernel(
    _sc_gather_body,
    out_type=(jax.ShapeDtypeStruct((E, H), jnp.float32),
              jax.ShapeDtypeStruct((E, H), jnp.float32)),
    mesh=_get_mesh(),
    scratch_types=[
        pltpu.VMEM((EPT,), jnp.int32),
        pltpu.VMEM((2 * KG, BG, H), jnp.float32),
        pltpu.SemaphoreType.DMA,
        pltpu.SemaphoreType.DMA,
        pltpu.SemaphoreType.DMA,
        pltpu.SemaphoreType.DMA,
    ],
  )


def _sc_gather_body(x_hbm, idx2_hbm, xs_hbm, xd_hbm, idx_v, rows_v,
                    gs0, gs1, ws0, ws1):
    c = lax.axis_index("c")
    s = lax.axis_index("s")
    pltpu.sync_copy(idx2_hbm.at[c, s], idx_v)
    base = s * EPT
    gsem = (gs0, gs1)
    wsem = (ws0, ws1)

    def run(out_hbm):
        def round_(rr, bank):
            j0 = (2 * rr + bank) * KG

            @pl.when(rr > 0)
            def _():
                for b in range(KG):
                    pltpu.make_async_copy(
                        x_hbm.at[pl.ds(0, BG)], rows_v.at[bank * KG + b],
                        wsem[bank]).wait()

            descs = [
                pltpu.async_copy(
                    x_hbm.at[idx_v.at[pl.ds((j0 + b) * BG, BG)]],
                    rows_v.at[bank * KG + b], gsem[bank])
                for b in range(KG)
            ]
            for d in descs:
                d.wait()
            for b in range(KG):
                pltpu.async_copy(
                    rows_v.at[bank * KG + b],
                    out_hbm.at[pl.ds(base + (j0 + b) * BG, BG)],
                    wsem[bank])

        def body(rr, carry):
            round_(rr, 0)
            round_(rr, 1)
            return carry

        lax.fori_loop(0, ITG, body, 0)
        for bank in range(2):
            for b in range(KG):
                pltpu.make_async_copy(
                    x_hbm.at[pl.ds(0, BG)], rows_v.at[bank * KG + b],
                    wsem[bank]).wait()

    @pl.when(c == 0)
    def _():
        run(xs_hbm)

    @pl.when(c == 1)
    def _():
        run(xd_hbm)


@functools.lru_cache(maxsize=None)
def _make_sc_scatter():
  return pl.kernel(
    _sc_scatter_body,
    out_type=(jax.ShapeDtypeStruct((NPAD, H), jnp.float32),
              jax.ShapeDtypeStruct((NPAD, H), jnp.float32)),
    mesh=_get_mesh(),
    scratch_types=[
        pltpu.VMEM((EPT + 16,), jnp.int32),
        pltpu.VMEM((NBANKS, BS), jnp.int32),
        pltpu.VMEM((NBANKS, BS, H), jnp.float32),
        pltpu.VMEM_SHARED((NPAD, H), jnp.float32),
    ] + [pltpu.SemaphoreType.DMA] * (2 * NBANKS),
  )


def _sc_scatter_body(xs_hbm, xd_hbm, comb2_hbm, zeros_hbm, mi_hbm, mo_hbm,
                     comb_v, idx_st, rows_v, acc_sh, *sems):
    c = lax.axis_index("c")
    s = lax.axis_index("s")
    gsem = sems[0:NBANKS]
    wsem = sems[NBANKS:2 * NBANKS]
    pltpu.sync_copy(comb2_hbm.at[c, s], comb_v)
    pltpu.sync_copy(zeros_hbm, acc_sh.at[pl.ds(s * ROWS_PT, ROWS_PT)])
    plsc.subcore_barrier()
    base = s * EPT

    def run(y_hbm):
        pltpu.async_copy(y_hbm.at[pl.ds(base, BS)], rows_v.at[0], gsem[0])
        pltpu.async_copy(y_hbm.at[pl.ds(base + BS, BS)], rows_v.at[1],
                         gsem[1])

        def slot(rr, bank):
            r = NBANKS * rr + bank
            nb = (bank + 2) % NBANKS

            def drain_w():
                pltpu.make_async_copy(
                    y_hbm.at[pl.ds(0, BS)], rows_v.at[nb], wsem[nb]).wait()

            def fire_g():
                pltpu.async_copy(
                    y_hbm.at[pl.ds(base + (r + 2) * BS, BS)],
                    rows_v.at[nb], gsem[nb])

            if bank < 2:
                @pl.when(rr > 0)
                def _():
                    drain_w()
                fire_g()
            else:
                drain_w()

                @pl.when(rr < ITS - 1)
                def _():
                    fire_g()
            for o in (0, 16, BS - 16):
                cv = comb_v[pl.ds(r * BS + o, 16)]
                idx_st[bank, pl.ds(o, 16)] = cv & jnp.int32(0x3FFF)
            pltpu.make_async_copy(
                y_hbm.at[pl.ds(0, BS)], rows_v.at[bank], gsem[bank]).wait()
            for o, lanes in ((0, range(0, 16)), (16, range(0, 16)),
                             (BS - 16, range(2 * 16 - (BS - 16), 16))):
                ev = comb_v[pl.ds(r * BS + o, 16)]
                evf = ((ev >> 14).astype(jnp.float32)
                       * jnp.float32(1.0 / 131071.0))
                for l in lanes:
                    esp = evf.at[jnp.full((16,), l, jnp.int32)].get(
                        mode="promise_in_bounds")
                    i = o + l
                    for k in range(8):
                        sl = pl.ds(k * 16, 16)
                        rows_v[bank, i, sl] = rows_v[bank, i, sl] * esp
            pltpu.async_copy(rows_v.at[bank], acc_sh.at[idx_st.at[bank]],
                             wsem[bank], add=True)

        def body(rr, carry):
            for bank in range(NBANKS):
                slot(rr, bank)
            return carry

        lax.fori_loop(0, ITS, body, 0)
        for bank in (2, 3):
            pltpu.make_async_copy(
                y_hbm.at[pl.ds(0, BS)], rows_v.at[bank], wsem[bank]).wait()

    @pl.when(c == 0)
    def _():
        run(xs_hbm)

    @pl.when(c == 1)
    def _():
        run(xd_hbm)

    plsc.subcore_barrier()

    def out_copy(out_hbm):
        pltpu.sync_copy(acc_sh.at[pl.ds(s * ROWS_PT, ROWS_PT)],
                        out_hbm.at[pl.ds(s * ROWS_PT, ROWS_PT)])

    @pl.when(c == 0)
    def _():
        out_copy(mi_hbm)

    @pl.when(c == 1)
    def _():
        out_copy(mo_hbm)


TN = 2000
TE = 2560

_full = lambda shape: pl.BlockSpec(shape, lambda i: (0,) * len(shape))


def _input_body(x_ref, w_ref, b_ref, o_ref):
    o_ref[...] = jnp.tanh(jnp.dot(x_ref[...], w_ref[...]) + b_ref[...])


def _input_mlp(x, W_in, b_in):
    return pl.pallas_call(
        _input_body,
        grid=(N // TN,),
        in_specs=[pl.BlockSpec((TN, H), lambda i: (i, 0)),
                  _full((H, H)), _full((1, H))],
        out_specs=pl.BlockSpec((TN, H), lambda i: (i, 0)),
        out_shape=jax.ShapeDtypeStruct((N, H), jnp.float32),
    )(x, W_in, b_in.reshape(1, H))


def _edge_body(xs_ref, xd_ref, w1a, w1b, b1, w2, b2, w3, b3, w4r, b4,
               e_ref):
    h = jnp.tanh(jnp.dot(xs_ref[...], w1a[...])
                 + jnp.dot(xd_ref[...], w1b[...]) + b1[...])
    h = jnp.tanh(jnp.dot(h, w2[...]) + b2[...])
    h = jnp.tanh(jnp.dot(h, w3[...]) + b3[...])
    logit = jnp.sum(h * w4r[...], axis=1, keepdims=True) + b4[...]
    e_ref[...] = jax.nn.sigmoid(logit)


_EDGE_W_SPECS = [
    _full((H, H)), _full((H, H)), _full((1, H)),
    _full((H, H)), _full((1, H)),
    _full((H, H)), _full((1, H)),
    _full((1, H)), _full((1, 1)),
]


def _edge_mlp(xs, xd, ew):
    espec = pl.BlockSpec((TE, 1), lambda i: (i, 0))
    rspec = pl.BlockSpec((TE, H), lambda i: (i, 0))
    return pl.pallas_call(
        _edge_body,
        grid=(E // TE,),
        in_specs=[rspec, rspec] + _EDGE_W_SPECS,
        out_specs=espec,
        out_shape=jax.ShapeDtypeStruct((E, 1), jnp.float32),
    )(xs, xd, *ew)


def _node_body(mi_ref, mo_ref, x_ref, w1a, w1b, w1c, b1, w2, b2, w3, b3,
               w4, b4, o_ref):
    x = x_ref[...]
    g = jnp.tanh(jnp.dot(mi_ref[...], w1a[...]) + jnp.dot(mo_ref[...], w1b[...])
                 + jnp.dot(x, w1c[...]) + b1[...])
    g = jnp.tanh(jnp.dot(g, w2[...]) + b2[...])
    g = jnp.tanh(jnp.dot(g, w3[...]) + b3[...])
    g = jnp.tanh(jnp.dot(g, w4[...]) + b4[...])
    o_ref[...] = x + g


def _node_mlp(mi, mo, x, nw):
    rspec = pl.BlockSpec((TN, H), lambda i: (i, 0))
    wspecs = [_full((H, H)), _full((H, H)), _full((H, H)), _full((1, H)),
              _full((H, H)), _full((1, H)), _full((H, H)), _full((1, H)),
              _full((H, H)), _full((1, H))]
    return pl.pallas_call(
        _node_body,
        grid=(N // TN,),
        in_specs=[rspec, rspec, rspec] + wspecs,
        out_specs=rspec,
        out_shape=jax.ShapeDtypeStruct((N, H), jnp.float32),
    )(mi, mo, x, *nw)


def kernel(x, edge_index, W_in, b_in, eW1, eb1, eW2, eb2, eW3, eb3, eW4, eb4,
           nW1, nb1, nW2, nb2, nW3, nb3, nW4, nb4):
    src = edge_index[0].astype(jnp.int32)
    dst = edge_index[1].astype(jnp.int32)
    idx_g = jnp.stack([src, dst]).reshape(2, NS, EPT)
    idx_s2 = jnp.stack([dst, src])
    zeros = jnp.zeros((ROWS_PT, H), jnp.float32)

    ew = (eW1[:H], eW1[H:], eb1.reshape(1, H), eW2, eb2.reshape(1, H),
          eW3, eb3.reshape(1, H), eW4.reshape(1, H), eb4.reshape(1, 1))
    nw = (nW1[:H], nW1[H:2 * H], nW1[2 * H:], nb1.reshape(1, H),
          nW2, nb2.reshape(1, H), nW3, nb3.reshape(1, H),
          nW4, nb4.reshape(1, H))

    xcur = _input_mlp(x, W_in, b_in)
    e = None
    for n in range(4):
        xs, xd = _make_sc_gather()(xcur, idx_g)
        e = _edge_mlp(xs, xd, ew)
        if n < 3:
            eint = jnp.round(e.reshape(E) * 131071.0).astype(jnp.int32)
            comb2 = jnp.pad((idx_s2 | (eint << 14)).reshape(2, NS, EPT),
                            ((0, 0), (0, 0), (0, 16)))
            mi, mo = _make_sc_scatter()(xs, xd, comb2, zeros)
            xcur = _node_mlp(mi, mo, xcur, nw)
    return e.reshape(E)
